# Initial kernel scaffold; baseline (speedup 1.0000x reference)
import jax
import jax.numpy as jnp
from jax.experimental import pallas as pl


def kernel(group_points, group_index, grasp, region_num):
    raise NotImplementedError("write your pallas kernel here")


# speedup vs baseline: 2.0074x; 2.0074x over previous
import functools

import jax
import jax.numpy as jnp
from jax import lax
from jax.experimental import pallas as pl
from jax.experimental.pallas import tpu as pltpu, tpu_sc as plsc

WIDTHS, HEIGHT, DEPTHS = 0.08, 0.02, 0.06
B, G, REGION = 1024, 2048, 512
NW = 32
GPW = B // NW
X_LIM = DEPTHS / 2.0
Y_LIM = WIDTHS / 2.0
Z_LIM = HEIGHT / 2.0

_mesh = plsc.VectorSubcoreMesh(core_axis_name="c", subcore_axis_name="s",
                               num_cores=2, num_subcores=16)


def _rne_jax(x):
    u = lax.bitcast_convert_type(x, jnp.uint32)
    r = (u + jnp.uint32(0x7FFF) + ((u >> 16) & jnp.uint32(1))) & jnp.uint32(0xFFFF0000)
    return lax.bitcast_convert_type(r, jnp.float32)


def _rne_sc(x):
    u = plsc.bitcast(x, jnp.int32)
    one = jnp.full((16,), 1, jnp.int32)
    rbit = lax.shift_right_logical(u, jnp.full((16,), 16, jnp.int32)) & one
    r = (u + jnp.full((16,), 0x7FFF, jnp.int32) + rbit) & jnp.full((16,), -65536, jnp.int32)
    return plsc.bitcast(r, jnp.float32)


def _frames(grasp):
    cx, cy, cz = grasp[:, 0], grasp[:, 1], grasp[:, 2]
    ayx, ayy, ayz = grasp[:, 3], grasp[:, 4], grasp[:, 5]
    angle = grasp[:, 6]
    c, s = jnp.cos(angle), jnp.sin(angle)
    ny = jnp.sqrt(ayx * ayx + ayy * ayy + ayz * ayz) + 1e-12
    ayx, ayy, ayz = ayx / ny, ayy / ny, ayz / ny
    nx = jnp.sqrt(ayy * ayy + ayx * ayx) + 1e-12
    axx, axy, axz = ayy / nx, -ayx / nx, jnp.zeros_like(ny)
    azx = axy * ayz - axz * ayy
    azy = axz * ayx - axx * ayz
    azz = axx * ayy - axy * ayx
    nz = jnp.sqrt(azx * azx + azy * azy + azz * azz)
    safe = jnp.where(nz == 0, 1.0, nz)
    azx = jnp.where(nz == 0, 0.0, azx / safe)
    azy = jnp.where(nz == 0, 0.0, azy / safe)
    azz = jnp.where(nz == 0, 1.0, azz / safe)
    cq, sq = _rne_jax(c), _rne_jax(s)
    apx = _rne_jax(axx) * cq + _rne_jax(azx) * sq
    apy = _rne_jax(axy) * cq + _rne_jax(azy) * sq
    apz = _rne_jax(axz) * cq + _rne_jax(azz) * sq
    na = jnp.sqrt(apx * apx + apy * apy + apz * apz) + 1e-12
    apx, apy, apz = apx / na, apy / na, apz / na
    mx = apy * ayz - apz * ayy
    my = apz * ayx - apx * ayz
    mz = apx * ayy - apy * ayx
    rows = [_rne_jax(v) for v in (apx, apy, apz, ayx, ayy, ayz, mx, my, mz)]
    return jnp.stack(rows + [cx, cy, cz], axis=1)


@functools.partial(
    pl.kernel,
    out_type=(
        jax.ShapeDtypeStruct((B, REGION * 6), jnp.float32),
        jax.ShapeDtypeStruct((B, REGION), jnp.int32),
        jax.ShapeDtypeStruct((B, REGION), jnp.int32),
        jax.ShapeDtypeStruct((B,), jnp.int32),
    ),
    mesh=_mesh,
    compiler_params=pltpu.CompilerParams(needs_layout_passes=False),
    scratch_types=[
        pltpu.VMEM((G * 6,), jnp.float32),
        pltpu.VMEM((G,), jnp.int32),
        pltpu.VMEM((GPW * 12,), jnp.float32),
        pltpu.VMEM((G + 16,), jnp.int32),
        pltpu.VMEM((REGION * 6,), jnp.float32),
        pltpu.VMEM((REGION,), jnp.int32),
        pltpu.VMEM((REGION,), jnp.int32),
        pltpu.VMEM((GPW,), jnp.int32),
    ],
)
def _region_sc(pts_hbm, gidx_hbm, fc_hbm, pc_hbm, idx_hbm, inall_hbm,
               valid_hbm, pts_v, gidx_v, fc_v, idxl_v, pc_v, idxo_v,
               inall_v, valid_v):
    wid = lax.axis_index("s") * 2 + lax.axis_index("c")
    base = wid * GPW
    pltpu.sync_copy(fc_hbm.at[pl.ds(base * 12, GPW * 12)], fc_v)
    lanes = lax.iota(jnp.int32, 16)

    def splat_i(v):
        return jnp.full((16,), v, jnp.int32)

    def per_grasp(gi, carry):
        g = base + gi
        pltpu.sync_copy(pts_hbm.at[g], pts_v)
        pltpu.sync_copy(gidx_hbm.at[g], gidx_v)
        fbase = splat_i(gi * 12)
        fv = [plsc.load_gather(fc_v, [fbase + splat_i(r)]) for r in range(12)]
        f00, f01, f02, f10, f11, f12, f20, f21, f22, cx, cy, cz = fv

        def transform(x, y, z):
            rx = _rne_sc(x - cx)
            ry = _rne_sc(y - cy)
            rz = _rne_sc(z - cz)
            t0 = f00 * rx + (f01 * ry + f02 * rz)
            t1 = f10 * rx + (f11 * ry + f12 * rz)
            t2 = f20 * rx + (f21 * ry + f22 * rz)
            return t0, t1, t2

        def body_a(i, cnt):
            r = lanes + i * 16
            r6 = r * 6
            x = plsc.load_gather(pts_v, [r6])
            y = plsc.load_gather(pts_v, [r6 + splat_i(1)])
            z = plsc.load_gather(pts_v, [r6 + splat_i(2)])
            t0, t1, t2 = transform(x, y, z)
            m = ((t0 > 0) & (t0 < X_LIM)
                 & (t1 > -Y_LIM) & (t1 < Y_LIM)
                 & (t2 > -Z_LIM) & (t2 < Z_LIM))
            plsc.store_compressed(idxl_v.at[pl.ds(cnt, 16)], r, mask=m)
            return cnt + jnp.sum(m.astype(jnp.int32))

        cnt = lax.fori_loop(0, G // 16, body_a, 0)
        cnt_s = splat_i(cnt)
        validv = cnt_s > 5
        plsc.store_scatter(valid_v, [splat_i(gi)],
                           jnp.where(validv, 1, 0).astype(jnp.int32),
                           mask=lanes == 0)
        denom = jnp.maximum(cnt_s, 1)
        neg1f = jnp.full((16,), -1.0, jnp.float32)
        neg1i = splat_i(-1)

        def body_b(j, carry2):
            pos = lanes + j * 16
            sel = plsc.load_gather(idxl_v, [lax.rem(pos, denom)])
            s6 = sel * 6
            x = plsc.load_gather(pts_v, [s6])
            y = plsc.load_gather(pts_v, [s6 + splat_i(1)])
            z = plsc.load_gather(pts_v, [s6 + splat_i(2)])
            fa = plsc.load_gather(pts_v, [s6 + splat_i(3)])
            fb = plsc.load_gather(pts_v, [s6 + splat_i(4)])
            fc = plsc.load_gather(pts_v, [s6 + splat_i(5)])
            t0, t1, t2 = transform(x, y, z)
            p6 = pos * 6
            plsc.store_scatter(pc_v, [p6], jnp.where(validv, t0, neg1f))
            plsc.store_scatter(pc_v, [p6 + splat_i(1)], jnp.where(validv, t1, neg1f))
            plsc.store_scatter(pc_v, [p6 + splat_i(2)], jnp.where(validv, t2, neg1f))
            plsc.store_scatter(pc_v, [p6 + splat_i(3)], jnp.where(validv, fa, neg1f))
            plsc.store_scatter(pc_v, [p6 + splat_i(4)], jnp.where(validv, fb, neg1f))
            plsc.store_scatter(pc_v, [p6 + splat_i(5)], jnp.where(validv, fc, neg1f))
            idxo_v[pl.ds(j * 16, 16)] = jnp.where(validv, sel, neg1i)
            ia = plsc.load_gather(gidx_v, [sel])
            inall_v[pl.ds(j * 16, 16)] = jnp.where(validv, ia, neg1i)
            return carry2

        lax.fori_loop(0, REGION // 16, body_b, 0)
        pltpu.sync_copy(pc_v, pc_hbm.at[g])
        pltpu.sync_copy(idxo_v, idx_hbm.at[g])
        pltpu.sync_copy(inall_v, inall_hbm.at[g])
        return carry

    lax.fori_loop(0, GPW, per_grasp, 0)
    pltpu.sync_copy(valid_v, valid_hbm.at[pl.ds(base, GPW)])


@functools.partial(
    pl.kernel,
    out_type=jax.ShapeDtypeStruct((B,), jnp.int32),
    mesh=_mesh,
    compiler_params=pltpu.CompilerParams(needs_layout_passes=False),
    scratch_types=[
        pltpu.VMEM((B,), jnp.int32),
        pltpu.VMEM((B + 16,), jnp.int32),
    ],
)
def _tmi_sc(valid_hbm, tmi_hbm, val_v, out_v):
    wid = lax.axis_index("s") * 2 + lax.axis_index("c")
    lanes = lax.iota(jnp.int32, 16)

    @pl.when(wid == 0)
    def _():
        pltpu.sync_copy(valid_hbm, val_v)
        neg1 = jnp.full((16,), -1, jnp.int32)

        def clear(i, c):
            out_v[pl.ds(i * 16, 16)] = neg1
            return c

        lax.fori_loop(0, B // 16, clear, 0)

        def body(i, cnt):
            m = val_v[pl.ds(i * 16, 16)] > 0
            plsc.store_compressed(out_v.at[pl.ds(cnt, 16)], lanes + i * 16,
                                  mask=m)
            return cnt + jnp.sum(m.astype(jnp.int32))

        lax.fori_loop(0, B // 16, body, 0)
        pltpu.sync_copy(out_v.at[pl.ds(0, B)], tmi_hbm)


def kernel(group_points, group_index, grasp, region_num):
    fc = _frames(grasp)
    pts_flat = group_points.reshape(B, G * 6)
    fc_flat = fc.reshape(B * 12)
    pc, idx, inall, valid = _region_sc(pts_flat, group_index, fc_flat)
    tmi = _tmi_sc(valid)
    return (pc.reshape(B, REGION, 6), idx, inall, tmi)
